# initial kernel scaffold (unmeasured)
import jax
import jax.numpy as jnp
from jax import lax
from jax.experimental import pallas as pl
from jax.experimental.pallas import tpu as pltpu

N_BLK = 4


def _ar_body(p_ref, out_ref, zrecv, send_sems, recv_sems):
    x = lax.axis_index("x")
    y = lax.axis_index("y")
    z = lax.axis_index("z")
    blk, d = p_ref.shape
    b = 2 * x + y
    bx = 2 * (1 - x) + y

    zswap = pltpu.make_async_remote_copy(
        src_ref=p_ref,
        dst_ref=zrecv,
        send_sem=send_sems.at[0],
        recv_sem=recv_sems.at[0],
        device_id=(x, y, 1 - z),
        device_id_type=pl.DeviceIdType.MESH,
    )
    zswap.start()
    zswap.wait()
    out_ref[pl.ds(b * blk, blk), :] = (
        p_ref[...].astype(jnp.float32) + zrecv[...].astype(jnp.float32)
    ).astype(jnp.bfloat16)

    xswap = pltpu.make_async_remote_copy(
        src_ref=out_ref.at[pl.ds(b * blk, blk)],
        dst_ref=out_ref.at[pl.ds(b * blk, blk)],
        send_sem=send_sems.at[1],
        recv_sem=recv_sems.at[1],
        device_id=(1 - x, y, z),
        device_id_type=pl.DeviceIdType.MESH,
    )
    xswap.start()
    xswap.wait()

    yswap0 = pltpu.make_async_remote_copy(
        src_ref=out_ref.at[pl.ds(b * blk, blk)],
        dst_ref=out_ref.at[pl.ds(b * blk, blk)],
        send_sem=send_sems.at[2],
        recv_sem=recv_sems.at[2],
        device_id=(x, 1 - y, z),
        device_id_type=pl.DeviceIdType.MESH,
    )
    yswap1 = pltpu.make_async_remote_copy(
        src_ref=out_ref.at[pl.ds(bx * blk, blk)],
        dst_ref=out_ref.at[pl.ds(bx * blk, blk)],
        send_sem=send_sems.at[3],
        recv_sem=recv_sems.at[3],
        device_id=(x, 1 - y, z),
        device_id_type=pl.DeviceIdType.MESH,
    )
    yswap0.start()
    yswap1.start()
    yswap0.wait()
    yswap1.wait()


def _all_reduce_gather(partial, m, d):
    blk = partial.shape[0]
    return pl.pallas_call(
        _ar_body,
        out_shape=jax.ShapeDtypeStruct((m, d), jnp.bfloat16),
        in_specs=[pl.BlockSpec(memory_space=pltpu.VMEM)],
        out_specs=pl.BlockSpec(memory_space=pltpu.VMEM),
        scratch_shapes=[
            pltpu.VMEM((blk, d), jnp.bfloat16),
            pltpu.SemaphoreType.DMA((4,)),
            pltpu.SemaphoreType.DMA((4,)),
        ],
    )(partial)


def kernel(dy, W):
    m, _ = dy.shape
    d = W.shape[0]
    blk = m // N_BLK
    x = lax.axis_index("x")
    y = lax.axis_index("y")
    b = 2 * x + y
    dyb = lax.dynamic_slice_in_dim(dy, b * blk, blk, axis=0)
    partial = lax.dot_general(
        dyb.astype(jnp.bfloat16),
        W.astype(jnp.bfloat16),
        dimension_numbers=(((1,), (1,)), ((), ())),
        preferred_element_type=jnp.float32,
    ).astype(jnp.bfloat16)
    return _all_reduce_gather(partial, m, d)


# baseline (device time: 498589 ns/iter reference)
import jax
import jax.numpy as jnp
from jax import lax
from jax.experimental import pallas as pl
from jax.experimental.pallas import tpu as pltpu

N_BLK = 4


def _ar_body(p_ref, out_ref, zrecv, send_sems, recv_sems, local_sem):
    x = lax.axis_index("x")
    y = lax.axis_index("y")
    z = lax.axis_index("z")
    blk, d = p_ref.shape
    b = 2 * x + y
    bx = 2 * (1 - x) + y

    zswap = pltpu.make_async_remote_copy(
        src_ref=p_ref,
        dst_ref=zrecv,
        send_sem=send_sems.at[0],
        recv_sem=recv_sems.at[0],
        device_id=(x, y, 1 - z),
        device_id_type=pl.DeviceIdType.MESH,
    )
    zswap.start()
    zswap.wait()
    zrecv[...] = (
        p_ref[...].astype(jnp.float32) + zrecv[...].astype(jnp.float32)
    ).astype(jnp.bfloat16)

    local_copy = pltpu.make_async_copy(
        zrecv, out_ref.at[pl.ds(b * blk, blk)], local_sem
    )
    local_copy.start()

    xswap = pltpu.make_async_remote_copy(
        src_ref=zrecv,
        dst_ref=out_ref.at[pl.ds(b * blk, blk)],
        send_sem=send_sems.at[1],
        recv_sem=recv_sems.at[1],
        device_id=(1 - x, y, z),
        device_id_type=pl.DeviceIdType.MESH,
    )
    xswap.start()
    xswap.wait()

    yswap0 = pltpu.make_async_remote_copy(
        src_ref=zrecv,
        dst_ref=out_ref.at[pl.ds(b * blk, blk)],
        send_sem=send_sems.at[2],
        recv_sem=recv_sems.at[2],
        device_id=(x, 1 - y, z),
        device_id_type=pl.DeviceIdType.MESH,
    )
    yswap1 = pltpu.make_async_remote_copy(
        src_ref=out_ref.at[pl.ds(bx * blk, blk)],
        dst_ref=out_ref.at[pl.ds(bx * blk, blk)],
        send_sem=send_sems.at[3],
        recv_sem=recv_sems.at[3],
        device_id=(x, 1 - y, z),
        device_id_type=pl.DeviceIdType.MESH,
    )
    yswap0.start()
    yswap1.start()
    yswap0.wait()
    yswap1.wait()
    local_copy.wait()


def _all_reduce_gather(partial, m, d):
    blk = partial.shape[0]
    return pl.pallas_call(
        _ar_body,
        out_shape=jax.ShapeDtypeStruct((m, d), jnp.bfloat16),
        in_specs=[pl.BlockSpec(memory_space=pltpu.VMEM)],
        out_specs=pl.BlockSpec(memory_space=pl.ANY),
        scratch_shapes=[
            pltpu.VMEM((blk, d), jnp.bfloat16),
            pltpu.SemaphoreType.DMA((4,)),
            pltpu.SemaphoreType.DMA((4,)),
            pltpu.SemaphoreType.DMA,
        ],
    )(partial)


def kernel(dy, W):
    m, _ = dy.shape
    d = W.shape[0]
    blk = m // N_BLK
    x = lax.axis_index("x")
    y = lax.axis_index("y")
    b = 2 * x + y
    dyb = lax.dynamic_slice_in_dim(dy, b * blk, blk, axis=0)
    partial = lax.dot_general(
        dyb.astype(jnp.bfloat16),
        W.astype(jnp.bfloat16),
        dimension_numbers=(((1,), (1,)), ((), ())),
        preferred_element_type=jnp.float32,
    ).astype(jnp.bfloat16)
    return _all_reduce_gather(partial, m, d)


# device time: 282010 ns/iter; 1.7680x vs baseline; 1.7680x over previous
import jax
import jax.numpy as jnp
from jax import lax
from jax.experimental import pallas as pl
from jax.experimental.pallas import tpu as pltpu

N_BLK = 4
N_CHUNK = 8

_MESH = pl.DeviceIdType.MESH


def _ar_body(
    p_ref, out_ref, zrecv,
    zs, zr, xs, xr, yos, yor, xfs, xfr, yfs, yfr, local_sem,
):
    x = lax.axis_index("x")
    y = lax.axis_index("y")
    z = lax.axis_index("z")
    blk, d = p_ref.shape
    ck = blk // N_CHUNK
    d2 = d // 2
    b = 2 * x + y
    bx = 2 * (1 - x) + y
    by = 2 * x + (1 - y)
    zdev = (x, y, 1 - z)
    xdev = (1 - x, y, z)
    ydev = (x, 1 - y, z)

    zswaps = []
    for c in range(N_CHUNK):
        rc = pl.ds(c * ck, ck)
        zc = pltpu.make_async_remote_copy(
            src_ref=p_ref.at[rc], dst_ref=zrecv.at[rc],
            send_sem=zs.at[c], recv_sem=zr.at[c],
            device_id=zdev, device_id_type=_MESH,
        )
        zc.start()
        zswaps.append(zc)

    xswaps = []
    yowns = []
    for c in range(N_CHUNK):
        rc = pl.ds(c * ck, ck)
        zswaps[c].wait_recv()
        zrecv[rc, :] = (
            p_ref[rc, :].astype(jnp.float32) + zrecv[rc, :].astype(jnp.float32)
        ).astype(jnp.bfloat16)
        rows_b_c = pl.ds(b * blk + c * ck, ck)
        xc = pltpu.make_async_remote_copy(
            src_ref=zrecv.at[rc], dst_ref=out_ref.at[rows_b_c],
            send_sem=xs.at[c], recv_sem=xr.at[c],
            device_id=xdev, device_id_type=_MESH,
        )
        xc.start()
        xswaps.append(xc)
        yc = pltpu.make_async_remote_copy(
            src_ref=zrecv.at[rc], dst_ref=out_ref.at[rows_b_c],
            send_sem=yos.at[c], recv_sem=yor.at[c],
            device_id=ydev, device_id_type=_MESH,
        )
        yc.start()
        yowns.append(yc)

    local = pltpu.make_async_copy(
        zrecv, out_ref.at[pl.ds(b * blk, blk)], local_sem
    )
    local.start()

    yfwds = []
    xfwds = []
    for c in range(N_CHUNK):
        xswaps[c].wait_recv()
        rows_bx_c = pl.ds(bx * blk + c * ck, ck)
        yf = pltpu.make_async_remote_copy(
            src_ref=out_ref.at[rows_bx_c, pl.ds(0, d2)],
            dst_ref=out_ref.at[rows_bx_c, pl.ds(0, d2)],
            send_sem=yfs.at[c], recv_sem=yfr.at[c],
            device_id=ydev, device_id_type=_MESH,
        )
        yf.start()
        yfwds.append(yf)
        yowns[c].wait_recv()
        rows_by_c = pl.ds(by * blk + c * ck, ck)
        xf = pltpu.make_async_remote_copy(
            src_ref=out_ref.at[rows_by_c, pl.ds(d2, d2)],
            dst_ref=out_ref.at[rows_by_c, pl.ds(d2, d2)],
            send_sem=xfs.at[c], recv_sem=xfr.at[c],
            device_id=xdev, device_id_type=_MESH,
        )
        xf.start()
        xfwds.append(xf)

    for c in range(N_CHUNK):
        zswaps[c].wait_send()
        xswaps[c].wait_send()
        yowns[c].wait_send()
        yfwds[c].wait_send()
        xfwds[c].wait_send()
        yfwds[c].wait_recv()
        xfwds[c].wait_recv()
    local.wait()


def _all_reduce_gather(partial, m, d):
    blk = partial.shape[0]
    sem = pltpu.SemaphoreType.DMA((N_CHUNK,))
    return pl.pallas_call(
        _ar_body,
        out_shape=jax.ShapeDtypeStruct((m, d), jnp.bfloat16),
        in_specs=[pl.BlockSpec(memory_space=pltpu.VMEM)],
        out_specs=pl.BlockSpec(memory_space=pl.ANY),
        scratch_shapes=[
            pltpu.VMEM((blk, d), jnp.bfloat16),
            sem, sem,
            sem, sem,
            sem, sem,
            sem, sem,
            sem, sem,
            pltpu.SemaphoreType.DMA,
        ],
    )(partial)


def kernel(dy, W):
    m, _ = dy.shape
    d = W.shape[0]
    blk = m // N_BLK
    x = lax.axis_index("x")
    y = lax.axis_index("y")
    b = 2 * x + y
    dyb = lax.dynamic_slice_in_dim(dy, b * blk, blk, axis=0)
    partial = lax.dot_general(
        dyb.astype(jnp.bfloat16),
        W.astype(jnp.bfloat16),
        dimension_numbers=(((1,), (1,)), ((), ())),
        preferred_element_type=jnp.float32,
    ).astype(jnp.bfloat16)
    return _all_reduce_gather(partial, m, d)
